# Initial kernel scaffold; baseline (speedup 1.0000x reference)
import functools

import jax
import jax.numpy as jnp
from jax import lax
from jax.experimental import pallas as pl
from jax.experimental.pallas import tpu as pltpu
from jax.experimental.pallas import tpu_sc as plsc

ZMAXPAD = 88
DIM = 64
N_ATOMS = 1048576

NC = 2
NS = 16
NW = NC * NS
B_PER_W = N_ATOMS // NW
CHUNK = 128
N_CHUNKS = B_PER_W // CHUNK


def kernel(species, table):
    mesh = plsc.VectorSubcoreMesh(core_axis_name="c", subcore_axis_name="s")

    @functools.partial(
        pl.kernel,
        mesh=mesh,
        out_type=jax.ShapeDtypeStruct((NW, N_CHUNKS, CHUNK, DIM), jnp.float32),
        scratch_types=[
            pltpu.VMEM((N_CHUNKS, CHUNK), jnp.int32),
            pltpu.VMEM((CHUNK, DIM), jnp.float32),
            pltpu.VMEM((CHUNK, DIM), jnp.float32),
            pltpu.SemaphoreType.DMA,
            pltpu.SemaphoreType.DMA,
        ],
    )
    def sc_gather(species_hbm, table_hbm, out_hbm, idx_v, rows0, rows1,
                  gsem0, gsem1):
        wid = lax.axis_index("s") * NC + lax.axis_index("c")
        pltpu.sync_copy(species_hbm.at[wid], idx_v)

        pltpu.async_copy(table_hbm.at[idx_v.at[0]], rows0, gsem0)

        def body(jj, _):
            j0 = 2 * jj
            pltpu.async_copy(table_hbm.at[idx_v.at[j0 + 1]], rows1, gsem1)
            pltpu.make_async_copy(table_hbm.at[idx_v.at[j0]], rows0,
                                  gsem0).wait()
            pltpu.sync_copy(rows0, out_hbm.at[wid, j0])

            @pl.when(j0 + 2 < N_CHUNKS)
            def _start_next():
                pltpu.async_copy(table_hbm.at[idx_v.at[j0 + 2]], rows0, gsem0)

            pltpu.make_async_copy(table_hbm.at[idx_v.at[j0 + 1]], rows1,
                                  gsem1).wait()
            pltpu.sync_copy(rows1, out_hbm.at[wid, j0 + 1])
            return None

        lax.fori_loop(0, N_CHUNKS // 2, body, None)

    species_blocked = species.reshape(NW, N_CHUNKS, CHUNK)
    out = sc_gather(species_blocked, table)
    return out.reshape(N_ATOMS, DIM)


# speedup vs baseline: 2.7747x; 2.7747x over previous
import functools

import jax
import jax.numpy as jnp
from jax import lax
from jax.experimental import pallas as pl
from jax.experimental.pallas import tpu as pltpu
from jax.experimental.pallas import tpu_sc as plsc

ZMAXPAD = 88
DIM = 64
N_ATOMS = 1048576

NC = 2
NS = 16
NW = NC * NS
B_PER_W = N_ATOMS // NW
CHUNK = 128
N_CHUNKS = B_PER_W // CHUNK


def kernel(species, table):
    mesh = plsc.VectorSubcoreMesh(core_axis_name="c", subcore_axis_name="s")

    @functools.partial(
        pl.kernel,
        mesh=mesh,
        compiler_params=pltpu.CompilerParams(use_tc_tiling_on_sc=False),
        out_type=jax.ShapeDtypeStruct((NW, N_CHUNKS, CHUNK, DIM), jnp.float32),
        scratch_types=[
            pltpu.VMEM((N_CHUNKS, CHUNK), jnp.int32),
            pltpu.VMEM((CHUNK, DIM), jnp.float32),
            pltpu.VMEM((CHUNK, DIM), jnp.float32),
            pltpu.SemaphoreType.DMA,
            pltpu.SemaphoreType.DMA,
        ],
    )
    def sc_gather(species_hbm, table_hbm, out_hbm, idx_v, rows0, rows1,
                  gsem0, gsem1):
        wid = lax.axis_index("s") * NC + lax.axis_index("c")
        pltpu.sync_copy(species_hbm.at[wid], idx_v)

        pltpu.async_copy(table_hbm.at[idx_v.at[0]], rows0, gsem0)

        def body(jj, _):
            j0 = 2 * jj
            pltpu.async_copy(table_hbm.at[idx_v.at[j0 + 1]], rows1, gsem1)
            pltpu.make_async_copy(table_hbm.at[idx_v.at[j0]], rows0,
                                  gsem0).wait()
            pltpu.sync_copy(rows0, out_hbm.at[wid, j0])

            @pl.when(j0 + 2 < N_CHUNKS)
            def _start_next():
                pltpu.async_copy(table_hbm.at[idx_v.at[j0 + 2]], rows0, gsem0)

            pltpu.make_async_copy(table_hbm.at[idx_v.at[j0 + 1]], rows1,
                                  gsem1).wait()
            pltpu.sync_copy(rows1, out_hbm.at[wid, j0 + 1])
            return None

        lax.fori_loop(0, N_CHUNKS // 2, body, None)

    species_blocked = species.reshape(NW, N_CHUNKS, CHUNK)
    out = sc_gather(species_blocked, table)
    return out.reshape(N_ATOMS, DIM)
